# baseline (speedup 1.0000x reference)
import jax
import jax.numpy as jnp
from jax import lax
from jax.experimental import pallas as pl
from jax.experimental.pallas import tpu as pltpu

_E = 64
_ZL = 0.001
_ROWS = 32768
_COLS = 2048
_BLK = 2048
_N = float(_ROWS * _COLS)


def _body(x_ref, p_ref, mask_ref, probs_ref, loss_ref, topk_ref, acc_ref):
    step = pl.program_id(0)

    @pl.when(step == 0)
    def _init():
        acc_ref[...] = jnp.zeros_like(acc_ref)

    acc_ref[...] += jnp.sum(x_ref[...], axis=0, keepdims=True)

    @pl.when(step == pl.num_programs(0) - 1)
    def _finish():
        total = jnp.sum(acc_ref[...], keepdims=True)
        x = total / _N
        p = p_ref[...]
        sim = -((p - x) ** 2)
        m = jnp.max(sim, keepdims=True)
        e = jnp.exp(sim - m)
        denom = jnp.sum(e, keepdims=True)
        probs = e / denom

        idx = lax.broadcasted_iota(jnp.int32, (1, _E), 1)
        m1 = jnp.max(probs, keepdims=True)
        i1 = jnp.min(jnp.where(probs == m1, idx, _E), keepdims=True)
        rest = jnp.where(idx == i1, -jnp.inf, probs)
        m2 = jnp.max(rest, keepdims=True)
        i2 = jnp.min(jnp.where(rest == m2, idx, _E), keepdims=True)

        mask_ref[...] = ((idx == i1) | (idx == i2)).astype(jnp.float32)
        probs_ref[...] = probs
        pm = jnp.sum(probs, keepdims=True) / _E
        loss_ref[...] = (pm - 1.0 / _E) ** 2 * _ZL
        k_iota = lax.broadcasted_iota(jnp.int32, (1, 2), 1)
        topk_ref[...] = jnp.where(k_iota == 0, i1, i2)


def kernel(wm_state, prototypes):
    wm = wm_state.reshape(_ROWS, _COLS)
    pt = prototypes.reshape(1, _E)
    grid = _ROWS // _BLK
    mask, probs, loss, topk = pl.pallas_call(
        _body,
        grid=(grid,),
        in_specs=[
            pl.BlockSpec((_BLK, _COLS), lambda i: (i, 0)),
            pl.BlockSpec((1, _E), lambda i: (0, 0)),
        ],
        out_specs=[
            pl.BlockSpec((1, _E), lambda i: (0, 0)),
            pl.BlockSpec((1, _E), lambda i: (0, 0)),
            pl.BlockSpec((1, 1), lambda i: (0, 0)),
            pl.BlockSpec((1, 2), lambda i: (0, 0)),
        ],
        out_shape=[
            jax.ShapeDtypeStruct((1, _E), jnp.float32),
            jax.ShapeDtypeStruct((1, _E), jnp.float32),
            jax.ShapeDtypeStruct((1, 1), jnp.float32),
            jax.ShapeDtypeStruct((1, 2), jnp.int32),
        ],
        scratch_shapes=[pltpu.VMEM((1, _COLS), jnp.float32)],
    )(wm, pt)
    return (mask.reshape(_E), probs.reshape(_E),
            loss.reshape(()), topk.reshape(2))


# speedup vs baseline: 1.0024x; 1.0024x over previous
import jax
import jax.numpy as jnp
from jax import lax
from jax.experimental import pallas as pl
from jax.experimental.pallas import tpu as pltpu

_E = 64
_ZL = 0.001
_ROWS = 32768
_COLS = 2048
_BLK = 1024
_HGRID = _ROWS // _BLK // 2
_N = float(_ROWS * _COLS)


def _body(x_ref, y_ref, p_ref, mask_ref, probs_ref, loss_ref, topk_ref, acc_ref):
    step = pl.program_id(0)

    @pl.when(step == 0)
    def _init():
        acc_ref[...] = jnp.zeros_like(acc_ref)

    acc_ref[...] += (jnp.sum(x_ref[...], axis=0, keepdims=True)
                     + jnp.sum(y_ref[...], axis=0, keepdims=True))

    @pl.when(step == pl.num_programs(0) - 1)
    def _finish():
        total = jnp.sum(acc_ref[...], keepdims=True)
        x = total / _N
        p = p_ref[...]
        sim = -((p - x) ** 2)
        m = jnp.max(sim, keepdims=True)
        e = jnp.exp(sim - m)
        denom = jnp.sum(e, keepdims=True)
        probs = e / denom

        idx = lax.broadcasted_iota(jnp.int32, (1, _E), 1)
        m1 = jnp.max(probs, keepdims=True)
        i1 = jnp.min(jnp.where(probs == m1, idx, _E), keepdims=True)
        rest = jnp.where(idx == i1, -jnp.inf, probs)
        m2 = jnp.max(rest, keepdims=True)
        i2 = jnp.min(jnp.where(rest == m2, idx, _E), keepdims=True)

        mask_ref[...] = ((idx == i1) | (idx == i2)).astype(jnp.float32)
        probs_ref[...] = probs
        pm = jnp.sum(probs, keepdims=True) / _E
        loss_ref[...] = (pm - 1.0 / _E) ** 2 * _ZL
        k_iota = lax.broadcasted_iota(jnp.int32, (1, 2), 1)
        topk_ref[...] = jnp.where(k_iota == 0, i1, i2)


def kernel(wm_state, prototypes):
    wm = wm_state.reshape(_ROWS, _COLS)
    pt = prototypes.reshape(1, _E)
    grid = _HGRID
    mask, probs, loss, topk = pl.pallas_call(
        _body,
        grid=(grid,),
        in_specs=[
            pl.BlockSpec((_BLK, _COLS), lambda i: (i, 0)),
            pl.BlockSpec((_BLK, _COLS), lambda i: (i + _HGRID, 0)),
            pl.BlockSpec((1, _E), lambda i: (0, 0)),
        ],
        out_specs=[
            pl.BlockSpec((1, _E), lambda i: (0, 0)),
            pl.BlockSpec((1, _E), lambda i: (0, 0)),
            pl.BlockSpec((1, 1), lambda i: (0, 0)),
            pl.BlockSpec((1, 2), lambda i: (0, 0)),
        ],
        out_shape=[
            jax.ShapeDtypeStruct((1, _E), jnp.float32),
            jax.ShapeDtypeStruct((1, _E), jnp.float32),
            jax.ShapeDtypeStruct((1, 1), jnp.float32),
            jax.ShapeDtypeStruct((1, 2), jnp.int32),
        ],
        scratch_shapes=[pltpu.VMEM((1, _COLS), jnp.float32)],
    )(wm, wm, pt)
    return (mask.reshape(_E), probs.reshape(_E),
            loss.reshape(()), topk.reshape(2))
